# baseline (speedup 1.0000x reference)
import jax
import jax.numpy as jnp
from jax import lax
from jax.experimental import pallas as pl
from jax.experimental.pallas import tpu as pltpu
from jax.experimental.pallas import tpu_sc as plsc

N_DIM = 128
G = 512
NC, NS = 2, 16
NW = NC * NS
CHUNK = 3200
STEPS = CHUNK // 16
NBINS = 640
NFEAT = 9
ACC = NFEAT * NBINS
MLP_BLOCK = 2048
BROWS = MLP_BLOCK // 128


def _fused_body(e_ref, w1_ref, w2_ref, b_ref, z_ref, p_ref,
                ox_ref, ob_ref, oz_ref, op_ref):
    a = jnp.dot(e_ref[...], w1_ref[...], preferred_element_type=jnp.float32)
    h = a / (1.0 + jnp.exp(-a))
    x = jnp.dot(h, w2_ref[...], preferred_element_type=jnp.float32)
    ox_ref[...] = x.reshape(BROWS, 128)
    ob_ref[...] = b_ref[...]
    oz_ref[...] = z_ref[...]
    op_ref[...] = p_ref[...]


def _fused_stage(e, w1, w2, b, z, p_flat):
    n = e.shape[0]
    grid = (n + MLP_BLOCK - 1) // MLP_BLOCK
    return pl.pallas_call(
        _fused_body,
        grid=(grid,),
        in_specs=[
            pl.BlockSpec((MLP_BLOCK, N_DIM), lambda i: (i, 0)),
            pl.BlockSpec((N_DIM, N_DIM), lambda i: (0, 0)),
            pl.BlockSpec((N_DIM, 1), lambda i: (0, 0)),
            pl.BlockSpec((MLP_BLOCK,), lambda i: (i,)),
            pl.BlockSpec((MLP_BLOCK,), lambda i: (i,)),
            pl.BlockSpec((MLP_BLOCK * 3,), lambda i: (i,)),
        ],
        out_specs=[
            pl.BlockSpec((BROWS, 128), lambda i: (i, 0)),
            pl.BlockSpec((MLP_BLOCK,), lambda i: (i,)),
            pl.BlockSpec((MLP_BLOCK,), lambda i: (i,)),
            pl.BlockSpec((MLP_BLOCK * 3,), lambda i: (i,)),
        ],
        out_shape=[
            jax.ShapeDtypeStruct((grid * BROWS, 128), jnp.float32),
            jax.ShapeDtypeStruct((n,), jnp.int32),
            jax.ShapeDtypeStruct((n,), jnp.int32),
            jax.ShapeDtypeStruct((n * 3,), jnp.float32),
        ],
    )(e, w1, w2, b, z, p_flat)


def _make_sc_body(n_total):
    def _sc_body(b_hbm, z_hbm, x_hbm, p_hbm, mass_hbm,
                 out_hbm, b_v, z_v, x_v, p_v, m_v, acc_v):
        wid = lax.axis_index("s") * NC + lax.axis_index("c")
        base = jnp.minimum(wid * CHUNK, n_total - CHUNK)
        base = pl.multiple_of(base, 32)
        skip = wid * CHUNK - base
        pltpu.sync_copy(b_hbm.at[pl.ds(base, CHUNK)],
                        b_v.at[pl.ds(0, CHUNK)])
        pltpu.sync_copy(z_hbm.at[pl.ds(base, CHUNK)], z_v)
        pltpu.sync_copy(x_hbm.at[pl.ds(base, CHUNK)], x_v)
        pltpu.sync_copy(p_hbm.at[pl.ds(base * 3, CHUNK * 3)], p_v)
        pltpu.sync_copy(mass_hbm, m_v)

        zero16 = jnp.zeros((16,), jnp.float32)

        def zstep(k, c):
            for u in range(8):
                acc_v[pl.ds(k * 128 + u * 16, 16)] = zero16
            return c

        lax.fori_loop(0, ACC // 128, zstep, 0)

        lanes = lax.iota(jnp.int32, 16)
        idx3 = lanes * 3

        def step(i, carry):
            off = i * 16
            b = b_v[pl.ds(off, 16)]
            b2 = b_v[pl.ds(off + 1, 16)]
            force = (lanes == 15) & (i == STEPS - 1)
            b2 = jnp.where(force, -1, b2)
            mask_p = b != b2
            mask_m = mask_p & (b2 >= 0)
            valid = (off + lanes) >= skip
            z = z_v[pl.ds(off, 16)]
            x = jnp.where(valid, x_v[pl.ds(off, 16)], 0.0)
            p_off = idx3 + off * 3
            px = plsc.load_gather(p_v, [p_off])
            py = plsc.load_gather(p_v, [p_off + 1])
            pz = plsc.load_gather(p_v, [p_off + 2])
            m = jnp.where(valid, plsc.load_gather(m_v, [z]), 0.0)
            p2 = px * px + py * py + pz * pz
            feats = (m, m * px, m * py, m * pz,
                     x, x * px, x * py, x * pz, x * p2)
            new_carry = []
            for f in range(NFEAT):
                cum = plsc.cumsum(feats[f]) + carry[f]
                plsc.addupdate_scatter(acc_v, [b + (f * NBINS)], cum,
                                       mask=mask_p)
                plsc.addupdate_scatter(acc_v, [b2 + (f * NBINS)], -cum,
                                       mask=mask_m)
                new_carry.append(cum[15])
            return tuple(new_carry)

        lax.fori_loop(0, STEPS, step, (jnp.float32(0.0),) * NFEAT)

        pltpu.sync_copy(acc_v, out_hbm.at[wid])

    return _sc_body


def _sc_segsums(b_t, z_t, x_flat, p_flat, m_pad):
    mesh = plsc.VectorSubcoreMesh(
        core_axis_name="c", subcore_axis_name="s",
        num_cores=NC, num_subcores=NS)
    return pl.kernel(
        _make_sc_body(b_t.shape[0]),
        out_type=jax.ShapeDtypeStruct((NW, ACC), jnp.float32),
        mesh=mesh,
        compiler_params=pltpu.CompilerParams(needs_layout_passes=False),
        scratch_types=[
            pltpu.VMEM((CHUNK + 16,), jnp.int32),
            pltpu.VMEM((CHUNK,), jnp.int32),
            pltpu.VMEM((CHUNK,), jnp.float32),
            pltpu.VMEM((CHUNK * 3,), jnp.float32),
            pltpu.VMEM((128,), jnp.float32),
            pltpu.VMEM((ACC,), jnp.float32),
        ],
    )(b_t, z_t, x_flat, p_flat, m_pad)


def _ep_body(acc_ref, o_ref):
    s = jnp.sum(acc_ref[...], axis=0)
    m0 = s[0:1, :]
    mx, my, mz = s[1:2, :], s[2:3, :], s[3:4, :]
    s0 = s[4:5, :]
    sx, sy, sz = s[5:6, :], s[6:7, :], s[7:8, :]
    s2 = s[8:9, :]
    den = jnp.where(m0 > 0.5, m0, 1.0)
    cx, cy, cz = mx / den, my / den, mz / den
    out = s2 - 2.0 * (cx * sx + cy * sy + cz * sz) \
        + (cx * cx + cy * cy + cz * cz) * s0
    o_ref[...] = out[:, :G]


def _epilogue(acc):
    return pl.pallas_call(
        _ep_body,
        out_shape=jax.ShapeDtypeStruct((1, G), jnp.float32),
    )(acc)


def kernel(energy, pos, masses, W1, W2, atomic_numbers, batch):
    x2, b_t, z_t, p_t = _fused_stage(
        energy, W1, W2, batch, atomic_numbers, pos.reshape(-1))
    m_pad = jnp.pad(masses, (0, 128 - masses.shape[0]), constant_values=1.0)
    acc = _sc_segsums(b_t, z_t, x2.reshape(-1), p_t, m_pad)
    out = _epilogue(acc.reshape(NW, NFEAT, NBINS))
    return out[0]


# speedup vs baseline: 1.3433x; 1.3433x over previous
import jax
import jax.numpy as jnp
from jax import lax
from jax.experimental import pallas as pl
from jax.experimental.pallas import tpu as pltpu
from jax.experimental.pallas import tpu_sc as plsc

N_DIM = 128
G = 512
NC, NS = 2, 16
NW = NC * NS
CHUNK = 3200
STEPS = CHUNK // 16
NBINS = 640
NFEAT = 9
ACC = NFEAT * NBINS
MLP_BLOCK = 2048
BROWS = MLP_BLOCK // 128


def _mlp_body(e_ref, w1_ref, w2_ref, o_ref):
    a = jnp.dot(e_ref[...], w1_ref[...], preferred_element_type=jnp.float32)
    h = a / (1.0 + jnp.exp(-a))
    x = jnp.dot(h, w2_ref[...], preferred_element_type=jnp.float32)
    o_ref[...] = x.reshape(BROWS, 128)


def _mlp(e, w1, w2):
    n = e.shape[0]
    grid = (n + MLP_BLOCK - 1) // MLP_BLOCK
    return pl.pallas_call(
        _mlp_body,
        grid=(grid,),
        in_specs=[
            pl.BlockSpec((MLP_BLOCK, N_DIM), lambda i: (i, 0)),
            pl.BlockSpec((N_DIM, N_DIM), lambda i: (0, 0)),
            pl.BlockSpec((N_DIM, 1), lambda i: (0, 0)),
        ],
        out_specs=pl.BlockSpec((BROWS, 128), lambda i: (i, 0)),
        out_shape=jax.ShapeDtypeStruct((grid * BROWS, 128), jnp.float32),
    )(e, w1, w2)


def _make_sc_body(n):
    m_off = 128
    b_off = m_off
    z_off = b_off + n
    x_off = z_off + n
    px_off = x_off + n
    py_off = px_off + n
    pz_off = py_off + n

    def _sc_body(t_hbm, out_hbm,
                 b_v, z_v, x_v, px_v, py_v, pz_v, m_v, acc_v):
        wid = lax.axis_index("s") * NC + lax.axis_index("c")
        base = jnp.minimum(wid * CHUNK, n - CHUNK)
        base = pl.multiple_of(base, 32)
        skip = wid * CHUNK - base
        pltpu.sync_copy(t_hbm.at[pl.ds(0, 128)], m_v)
        pltpu.sync_copy(t_hbm.at[pl.ds(b_off + base, CHUNK)],
                        b_v.at[pl.ds(0, CHUNK)])
        pltpu.sync_copy(t_hbm.at[pl.ds(z_off + base, CHUNK)], z_v)
        pltpu.sync_copy(t_hbm.at[pl.ds(x_off + base, CHUNK)], x_v)
        pltpu.sync_copy(t_hbm.at[pl.ds(px_off + base, CHUNK)], px_v)
        pltpu.sync_copy(t_hbm.at[pl.ds(py_off + base, CHUNK)], py_v)
        pltpu.sync_copy(t_hbm.at[pl.ds(pz_off + base, CHUNK)], pz_v)

        zero16 = jnp.zeros((16,), jnp.float32)

        def zstep(k, c):
            for u in range(8):
                acc_v[pl.ds(k * 128 + u * 16, 16)] = zero16
            return c

        lax.fori_loop(0, ACC // 128, zstep, 0)

        lanes = lax.iota(jnp.int32, 16)

        def step(i, carry):
            off = i * 16
            b = plsc.bitcast(b_v[pl.ds(off, 16)], jnp.int32)
            b2 = plsc.bitcast(b_v[pl.ds(off + 1, 16)], jnp.int32)
            force = (lanes == 15) & (i == STEPS - 1)
            b2 = jnp.where(force, -1, b2)
            mask_p = b != b2
            mask_m = mask_p & (b2 >= 0)
            valid = (off + lanes) >= skip
            z = plsc.bitcast(z_v[pl.ds(off, 16)], jnp.int32)
            x = jnp.where(valid, x_v[pl.ds(off, 16)], 0.0)
            px = px_v[pl.ds(off, 16)]
            py = py_v[pl.ds(off, 16)]
            pz = pz_v[pl.ds(off, 16)]
            m = jnp.where(valid, plsc.load_gather(m_v, [z]), 0.0)
            p2 = px * px + py * py + pz * pz
            feats = (m, m * px, m * py, m * pz,
                     x, x * px, x * py, x * pz, x * p2)
            new_carry = []
            for f in range(NFEAT):
                cum = plsc.cumsum(feats[f]) + carry[f]
                plsc.addupdate_scatter(acc_v, [b + (f * NBINS)], cum,
                                       mask=mask_p)
                plsc.addupdate_scatter(acc_v, [b2 + (f * NBINS)], -cum,
                                       mask=mask_m)
                new_carry.append(cum[15])
            return tuple(new_carry)

        lax.fori_loop(0, STEPS, step, (jnp.float32(0.0),) * NFEAT)

        pltpu.sync_copy(acc_v, out_hbm.at[wid])

    return _sc_body


def _sc_segsums(t, n):
    mesh = plsc.VectorSubcoreMesh(
        core_axis_name="c", subcore_axis_name="s",
        num_cores=NC, num_subcores=NS)
    return pl.kernel(
        _make_sc_body(n),
        out_type=jax.ShapeDtypeStruct((NW, ACC), jnp.float32),
        mesh=mesh,
        compiler_params=pltpu.CompilerParams(needs_layout_passes=False),
        scratch_types=[
            pltpu.VMEM((CHUNK + 16,), jnp.float32),
            pltpu.VMEM((CHUNK,), jnp.float32),
            pltpu.VMEM((CHUNK,), jnp.float32),
            pltpu.VMEM((CHUNK,), jnp.float32),
            pltpu.VMEM((CHUNK,), jnp.float32),
            pltpu.VMEM((CHUNK,), jnp.float32),
            pltpu.VMEM((128,), jnp.float32),
            pltpu.VMEM((ACC,), jnp.float32),
        ],
    )(t)


def _ep_body(acc_ref, o_ref):
    s = jnp.sum(acc_ref[...], axis=0)
    m0 = s[0:1, :]
    mx, my, mz = s[1:2, :], s[2:3, :], s[3:4, :]
    s0 = s[4:5, :]
    sx, sy, sz = s[5:6, :], s[6:7, :], s[7:8, :]
    s2 = s[8:9, :]
    den = jnp.where(m0 > 0.5, m0, 1.0)
    cx, cy, cz = mx / den, my / den, mz / den
    out = s2 - 2.0 * (cx * sx + cy * sy + cz * sz) \
        + (cx * cx + cy * cy + cz * cz) * s0
    o_ref[...] = out[:, :G]


def _epilogue(acc):
    return pl.pallas_call(
        _ep_body,
        out_shape=jax.ShapeDtypeStruct((1, G), jnp.float32),
    )(acc)


def kernel(energy, pos, masses, W1, W2, atomic_numbers, batch):
    n = energy.shape[0]
    x_flat = _mlp(energy, W1, W2).reshape(-1)[:n]
    t = jnp.concatenate([
        jnp.pad(masses, (0, 128 - masses.shape[0]), constant_values=1.0),
        jax.lax.bitcast_convert_type(batch, jnp.float32),
        jax.lax.bitcast_convert_type(atomic_numbers, jnp.float32),
        x_flat,
        pos[:, 0], pos[:, 1], pos[:, 2],
    ])
    acc = _sc_segsums(t, n)
    out = _epilogue(acc.reshape(NW, NFEAT, NBINS))
    return out[0]
